# Initial kernel scaffold; baseline (speedup 1.0000x reference)
import jax
import jax.numpy as jnp
from jax.experimental import pallas as pl


def kernel(X, images):
    raise NotImplementedError("write your pallas kernel here")


# speedup vs baseline: 117.8058x; 117.8058x over previous
import jax
import jax.numpy as jnp
from jax import lax
from jax.experimental import pallas as pl
from jax.experimental.pallas import tpu as pltpu

_C, _H, _W = 4, 512, 512
_EH, _EW = 64, 64
_NP = 512
_NPIC = 32


def _composite_fill_kernel(nvalid_ref, x1_ref, y1_ref, id_ref, images_ref,
                           out_ref, canvas_ref, sprite_ref, sem):
    pic = pl.program_id(0)

    @pl.when(pic < _NPIC - 1)
    def _fill():
        out_ref[...] = jnp.ones_like(out_ref)

    @pl.when(pic == _NPIC - 1)
    def _composite():
        canvas_ref[...] = jnp.ones_like(canvas_ref)
        _SH = _EH + 8

        def body(k, carry):
            x1 = x1_ref[k]
            y1 = y1_ref[k]
            eid = id_ref[k]
            cp = pltpu.make_async_copy(images_ref.at[eid], sprite_ref, sem)
            cp.start()
            cp.wait()
            ax = jnp.minimum((x1 // 8) * 8, _H - _SH)
            ax = pl.multiple_of(ax, 8)
            ox = x1 - ax
            strip = canvas_ref[:, pl.ds(ax, _SH), :]
            spr = sprite_ref[...]
            placed = jnp.concatenate(
                [spr, jnp.zeros((_C, _EH, _W - _EW), jnp.float32)], axis=2)
            placed = jnp.concatenate(
                [placed, jnp.zeros((_C, _SH - _EH, _W), jnp.float32)], axis=1)
            placed = pltpu.roll(placed, ox, axis=1)
            placed = pltpu.roll(placed, y1, axis=2)
            rows = ax + lax.broadcasted_iota(jnp.int32, (_SH, _W), 0)
            cols = lax.broadcasted_iota(jnp.int32, (_SH, _W), 1)
            inw = ((rows >= x1) & (rows < x1 + _EH)
                   & (cols >= y1) & (cols < y1 + _EW))
            a_new = jnp.where(inw, placed[3], 0.0)
            a_old = strip[3]
            a0 = a_new + a_old * (1.0 - a_new)
            rgb = (placed[:3] * a_new + strip[:3] * (a_old * (1.0 - a_new))) / a0
            blended = jnp.concatenate([rgb, a0[None]], axis=0)
            canvas_ref[:, pl.ds(ax, _SH), :] = jnp.where(inw[None], blended, strip)
            return carry

        lax.fori_loop(0, nvalid_ref[0], body, 0)
        out_ref[0] = canvas_ref[:3]


def kernel(X, images):
    data = X[-1]
    x = jnp.round(data[:, 0] * _H).astype(jnp.int32)
    y = jnp.round(data[:, 1] * _W).astype(jnp.int32)
    h = jnp.round(data[:, 2] * _H).astype(jnp.int32)
    w = jnp.round(data[:, 3] * _W).astype(jnp.int32)
    d = data[:, 4]
    c = data[:, 5:]

    x1 = x - h // 2
    y1 = y - w // 2
    x2 = x + (h + 1) // 2
    y2 = y + (w + 1) // 2
    valid = (x1 >= 0) & (y1 >= 0) & (x2 <= _H) & (y2 <= _W)
    eid = jnp.argmax(c, axis=1).astype(jnp.int32)
    x1c = jnp.clip(x1, 0, _H - _EH)
    y1c = jnp.clip(y1, 0, _W - _EW)

    key = jnp.where(valid, d, jnp.inf)
    order = jnp.argsort(key, stable=True)
    n_valid = jnp.sum(valid).astype(jnp.int32).reshape((1,))
    sx1 = x1c[order]
    sy1 = y1c[order]
    sid = eid[order]

    out = pl.pallas_call(
        _composite_fill_kernel,
        grid=(_NPIC,),
        in_specs=[
            pl.BlockSpec(memory_space=pltpu.SMEM),
            pl.BlockSpec(memory_space=pltpu.SMEM),
            pl.BlockSpec(memory_space=pltpu.SMEM),
            pl.BlockSpec(memory_space=pltpu.SMEM),
            pl.BlockSpec(memory_space=pl.ANY),
        ],
        out_specs=pl.BlockSpec((1, 3, _H, _W), lambda i: (i, 0, 0, 0)),
        out_shape=jax.ShapeDtypeStruct((_NPIC, 3, _H, _W), jnp.float32),
        scratch_shapes=[
            pltpu.VMEM((_C, _H, _W), jnp.float32),
            pltpu.VMEM((_C, _EH, _EW), jnp.float32),
            pltpu.SemaphoreType.DMA,
        ],
    )(n_valid, sx1, sy1, sid, images)
    return out
